# Initial kernel scaffold; baseline (speedup 1.0000x reference)
import jax
import jax.numpy as jnp
from jax.experimental import pallas as pl


def kernel(x, edge_index, batch, W0, b0, W1, b1, W2, b2, Wl1, bl1, Wl2, bl2):
    raise NotImplementedError("write your pallas kernel here")


# speedup vs baseline: 11.6395x; 11.6395x over previous
import functools

import jax
import jax.numpy as jnp
from jax import lax
from jax.experimental import pallas as pl
from jax.experimental.pallas import tpu as pltpu
from jax.experimental.pallas import tpu_sc as plsc

N_NODES = 10000
N_EDGES = 320000
IN_CH = 128
HID = 256
NUM_GRAPHS = 64

NC = 2
NS = 16
K = 125
EPT = N_EDGES // NS
RPT = EPT // K
EPT32 = N_EDGES // (NC * NS)
RPT32 = EPT32 // K
NP = 10240
STR = NP // NS

_MESH = plsc.VectorSubcoreMesh(core_axis_name="c", subcore_axis_name="s")
_SC_PARAMS = pltpu.CompilerParams(use_tc_tiling_on_sc=False)

ROWS_TC = 1000
GRID_TC = N_NODES // ROWS_TC


@functools.partial(
    pl.kernel,
    out_type=jax.ShapeDtypeStruct((NC, NP, 16), jnp.float32),
    mesh=_MESH,
    scratch_types=[
        pltpu.VMEM((RPT32, K), jnp.int32),
        pltpu.VMEM((K, 16), jnp.float32),
        pltpu.VMEM_SHARED((NP, 16), jnp.float32),
    ],
    compiler_params=_SC_PARAMS,
)
def _deg_kernel(dst_hbm, ones_hbm, zeros_hbm, degp_hbm, idx_v, ones_v, acc):
  c = lax.axis_index("c")
  s = lax.axis_index("s")
  w = c * NS + s
  pltpu.sync_copy(zeros_hbm, acc.at[pl.ds(s * STR, STR), :])
  pltpu.sync_copy(ones_hbm, ones_v)
  pltpu.sync_copy(dst_hbm.at[pl.ds(w * RPT32, RPT32), :], idx_v)
  plsc.subcore_barrier()

  def body(j, carry):
    pltpu.sync_copy(ones_v, acc.at[idx_v.at[j]], add=True)
    return carry

  lax.fori_loop(0, RPT32, body, 0)
  plsc.subcore_barrier()
  pltpu.sync_copy(acc.at[pl.ds(s * STR, STR), :],
                  degp_hbm.at[c, pl.ds(s * STR, STR), :])


def _make_agg_kernel(dh):
  @functools.partial(
      pl.kernel,
      out_type=jax.ShapeDtypeStruct((NC, NP, dh), jnp.float32),
      mesh=_MESH,
      scratch_types=[
          pltpu.VMEM((RPT, K), jnp.int32),
          pltpu.VMEM((RPT, K), jnp.int32),
          pltpu.VMEM((K, dh), jnp.float32),
          pltpu.VMEM_SHARED((NP, dh), jnp.float32),
          pltpu.SemaphoreType.DMA,
      ],
      compiler_params=_SC_PARAMS,
  )
  def agg_kernel(utab_hbm, src2_hbm, dst_hbm, zeros_hbm, agg_hbm,
                 src_v, dst_v, rows_v, acc, sem):
    c = lax.axis_index("c")
    s = lax.axis_index("s")
    pltpu.sync_copy(zeros_hbm, acc.at[pl.ds(s * STR, STR), :])
    pltpu.sync_copy(src2_hbm.at[c, pl.ds(s * RPT, RPT), :], src_v)
    pltpu.sync_copy(dst_hbm.at[pl.ds(s * RPT, RPT), :], dst_v)
    plsc.subcore_barrier()

    def body(j, carry):
      pltpu.async_copy(utab_hbm.at[src_v.at[j]], rows_v, sem).wait()
      pltpu.sync_copy(rows_v, acc.at[dst_v.at[j]], add=True)
      return carry

    lax.fori_loop(0, RPT, body, 0)
    plsc.subcore_barrier()
    pltpu.sync_copy(acc.at[pl.ds(s * STR, STR), :],
                    agg_hbm.at[c, pl.ds(s * STR, STR), :])

  return agg_kernel


_agg = _make_agg_kernel(64)


def _prep_body(degp_ref, x_ref, dinv_ref, u0_ref):
  d = degp_ref[:]
  deg = d[0, :, 0:1] + d[1, :, 0:1] + 1.0
  dinv = lax.rsqrt(deg)
  dinv_ref[:] = dinv
  u = x_ref[:] * dinv
  u0_ref[0] = u[:, : IN_CH // 2]
  u0_ref[1] = u[:, IN_CH // 2:]


def _prep_call(degp, x):
  return pl.pallas_call(
      _prep_body,
      grid=(GRID_TC,),
      in_specs=[
          pl.BlockSpec((NC, ROWS_TC, 16), lambda i: (0, i, 0)),
          pl.BlockSpec((ROWS_TC, IN_CH), lambda i: (i, 0)),
      ],
      out_specs=[
          pl.BlockSpec((ROWS_TC, 1), lambda i: (i, 0)),
          pl.BlockSpec((NC, ROWS_TC, IN_CH // 2), lambda i: (0, i, 0)),
      ],
      out_shape=[
          jax.ShapeDtypeStruct((N_NODES, 1), jnp.float32),
          jax.ShapeDtypeStruct((NC, N_NODES, IN_CH // 2), jnp.float32),
      ],
  )(degp, x)


def _make_layer_body(nplanes):
  def body(*refs):
    agg_refs = refs[:nplanes // 2]
    u_ref, dinv_ref, w_ref, b_ref, out_ref = refs[nplanes // 2:]
    parts = []
    for p in range(nplanes):
      parts.append(agg_refs[p // 2][p % 2] + u_ref[p])
    a = jnp.concatenate(parts, axis=1)
    dinv = dinv_ref[:]
    am = a * dinv
    h = jnp.dot(am, w_ref[:], preferred_element_type=jnp.float32) + b_ref[:]
    un = jnp.maximum(h, 0.0) * dinv
    for q in range(4):
      out_ref[q] = un[:, 64 * q: 64 * (q + 1)]
  return body


def _layer_call(aggs, u, dinv, w, b):
  nplanes = u.shape[0]
  din = 64 * nplanes
  agg_specs = [pl.BlockSpec((NC, ROWS_TC, 64), lambda i: (0, i, 0))
               for _ in aggs]
  return pl.pallas_call(
      _make_layer_body(nplanes),
      grid=(GRID_TC,),
      in_specs=agg_specs + [
          pl.BlockSpec((nplanes, ROWS_TC, 64), lambda i: (0, i, 0)),
          pl.BlockSpec((ROWS_TC, 1), lambda i: (i, 0)),
          pl.BlockSpec((din, HID), lambda i: (0, 0)),
          pl.BlockSpec((1, HID), lambda i: (0, 0)),
      ],
      out_specs=pl.BlockSpec((4, ROWS_TC, 64), lambda i: (0, i, 0)),
      out_shape=jax.ShapeDtypeStruct((4, N_NODES, 64), jnp.float32),
  )(*aggs, u, dinv, w, b.reshape(1, HID))


def _final_body(agga_ref, aggb_ref, u_ref, dinv_ref, w_ref, b_ref, batch_ref,
                wl1_ref, bl1_ref, wl2_ref, bl2_ref, out_ref, g_ref):
  i = pl.program_id(0)

  @pl.when(i == 0)
  def _():
    g_ref[:] = jnp.zeros_like(g_ref)

  agg_refs = (agga_ref, aggb_ref)
  parts = [agg_refs[p // 2][p % 2] + u_ref[p] for p in range(4)]
  a = jnp.concatenate(parts, axis=1)
  dinv = dinv_ref[:]
  am = a * dinv
  h3 = jnp.dot(am, w_ref[:], preferred_element_type=jnp.float32) + b_ref[:]
  ids = lax.broadcasted_iota(jnp.int32, (NUM_GRAPHS, ROWS_TC), 0)
  onehot = (ids == batch_ref[0]).astype(jnp.float32)
  g_ref[:] += jnp.dot(onehot, h3, preferred_element_type=jnp.float32)

  @pl.when(i == GRID_TC - 1)
  def _():
    y = jnp.maximum(
        jnp.dot(g_ref[:], wl1_ref[:], preferred_element_type=jnp.float32)
        + bl1_ref[:], 0.0)
    out_ref[:] = (
        jnp.dot(y, wl2_ref[:], preferred_element_type=jnp.float32)
        + bl2_ref[:])


def _final_call(agga, aggb, u, dinv, w2, b2, batch2, wl1, bl1, wl2, bl2):
  return pl.pallas_call(
      _final_body,
      grid=(GRID_TC,),
      in_specs=[
          pl.BlockSpec((NC, ROWS_TC, 64), lambda i: (0, i, 0)),
          pl.BlockSpec((NC, ROWS_TC, 64), lambda i: (0, i, 0)),
          pl.BlockSpec((4, ROWS_TC, 64), lambda i: (0, i, 0)),
          pl.BlockSpec((ROWS_TC, 1), lambda i: (i, 0)),
          pl.BlockSpec((HID, HID), lambda i: (0, 0)),
          pl.BlockSpec((1, HID), lambda i: (0, 0)),
          pl.BlockSpec((1, 1, ROWS_TC), lambda i: (i, 0, 0)),
          pl.BlockSpec((HID, 64), lambda i: (0, 0)),
          pl.BlockSpec((1, 64), lambda i: (0, 0)),
          pl.BlockSpec((64, 1), lambda i: (0, 0)),
          pl.BlockSpec((1, 1), lambda i: (0, 0)),
      ],
      out_specs=pl.BlockSpec((NUM_GRAPHS, 1), lambda i: (0, 0)),
      out_shape=jax.ShapeDtypeStruct((NUM_GRAPHS, 1), jnp.float32),
      scratch_shapes=[pltpu.VMEM((NUM_GRAPHS, HID), jnp.float32)],
  )(agga, aggb, u, dinv, w2, b2.reshape(1, HID), batch2,
    wl1, bl1.reshape(1, 64), wl2, bl2.reshape(1, 1))


def kernel(x, edge_index, batch, W0, b0, W1, b1, W2, b2, Wl1, bl1, Wl2, bl2):
  src = edge_index[0].astype(jnp.int32)
  dst = edge_index[1].astype(jnp.int32)
  src2 = jnp.stack([src, src + N_NODES]).reshape(NC, N_EDGES // K, K)
  dst_r = dst.reshape(N_EDGES // K, K)
  batch2 = batch.astype(jnp.int32).reshape(GRID_TC, 1, ROWS_TC)

  ones16 = jnp.ones((K, 16), jnp.float32)
  zeros16 = jnp.zeros((STR, 16), jnp.float32)
  zeros64 = jnp.zeros((STR, 64), jnp.float32)

  degp = _deg_kernel(dst_r, ones16, zeros16)
  dinv, u0 = _prep_call(degp, x)

  agg0 = _agg(u0.reshape(NC * N_NODES, 64), src2, dst_r, zeros64)
  u1 = _layer_call([agg0], u0, dinv, W0, b0)

  agg1a = _agg(u1[:2].reshape(NC * N_NODES, 64), src2, dst_r, zeros64)
  agg1b = _agg(u1[2:].reshape(NC * N_NODES, 64), src2, dst_r, zeros64)
  u2 = _layer_call([agg1a, agg1b], u1, dinv, W1, b1)

  agg2a = _agg(u2[:2].reshape(NC * N_NODES, 64), src2, dst_r, zeros64)
  agg2b = _agg(u2[2:].reshape(NC * N_NODES, 64), src2, dst_r, zeros64)
  return _final_call(agg2a, agg2b, u2, dinv, W2, b2, batch2,
                     Wl1, bl1, Wl2, bl2)


# speedup vs baseline: 22.7224x; 1.9522x over previous
import functools

import jax
import jax.numpy as jnp
from jax import lax
from jax.experimental import pallas as pl
from jax.experimental.pallas import tpu as pltpu
from jax.experimental.pallas import tpu_sc as plsc

N_NODES = 10000
N_EDGES = 320000
IN_CH = 128
HID = 256
NUM_GRAPHS = 64

NC = 2
NS = 16
K = 125
EPT = N_EDGES // NS
RPT = EPT // K
EPT32 = N_EDGES // (NC * NS)
RPT32 = EPT32 // K
NP = 10240
STR = NP // NS
NBUF = 4

_MESH = plsc.VectorSubcoreMesh(core_axis_name="c", subcore_axis_name="s")
_SC_PARAMS = pltpu.CompilerParams(use_tc_tiling_on_sc=False)

ROWS_TC = 1000
GRID_TC = N_NODES // ROWS_TC


@functools.partial(
    pl.kernel,
    out_type=jax.ShapeDtypeStruct((NC, NP, 16), jnp.float32),
    mesh=_MESH,
    scratch_types=[
        pltpu.VMEM((RPT32, K), jnp.int32),
        pltpu.VMEM((K, 16), jnp.float32),
        pltpu.VMEM_SHARED((NP, 16), jnp.float32),
        pltpu.SemaphoreType.DMA,
    ],
    compiler_params=_SC_PARAMS,
)
def _deg_kernel(dst_hbm, ones_hbm, zeros_hbm, degp_hbm, idx_v, ones_v, acc,
                dsem):
  c = lax.axis_index("c")
  s = lax.axis_index("s")
  w = c * NS + s
  pltpu.sync_copy(zeros_hbm, acc.at[pl.ds(s * STR, STR), :])
  pltpu.sync_copy(ones_hbm, ones_v)
  pltpu.sync_copy(dst_hbm.at[pl.ds(w * RPT32, RPT32), :], idx_v)
  plsc.subcore_barrier()

  def body(j, carry):
    pltpu.async_copy(ones_v, acc.at[idx_v.at[j]], dsem, add=True)
    return carry

  lax.fori_loop(0, RPT32, body, 0)

  def drain(j, carry):
    pltpu.make_async_copy(ones_v, acc.at[idx_v.at[0]], dsem).wait()
    return carry

  lax.fori_loop(0, RPT32, drain, 0)
  plsc.subcore_barrier()
  pltpu.sync_copy(acc.at[pl.ds(s * STR, STR), :],
                  degp_hbm.at[c, pl.ds(s * STR, STR), :])


def _make_agg_kernel(dh):
  @functools.partial(
      pl.kernel,
      out_type=jax.ShapeDtypeStruct((NC, NP, dh), jnp.float32),
      mesh=_MESH,
      scratch_types=[
          pltpu.VMEM((RPT, K), jnp.int32),
          pltpu.VMEM((RPT, K), jnp.int32),
          pltpu.VMEM((NBUF, K, dh), jnp.float32),
          pltpu.VMEM_SHARED((NP, dh), jnp.float32),
      ] + [pltpu.SemaphoreType.DMA] * NBUF,
      compiler_params=_SC_PARAMS,
  )
  def agg_kernel(utab_hbm, src2_hbm, dst_hbm, zeros_hbm, agg_hbm,
                 src_v, dst_v, rows_v, acc, *sems):
    c = lax.axis_index("c")
    s = lax.axis_index("s")
    pltpu.sync_copy(zeros_hbm, acc.at[pl.ds(s * STR, STR), :])
    pltpu.sync_copy(src2_hbm.at[c, pl.ds(s * RPT, RPT), :], src_v)
    pltpu.sync_copy(dst_hbm.at[pl.ds(s * RPT, RPT), :], dst_v)
    plsc.subcore_barrier()

    def fire(j, b):
      pltpu.async_copy(utab_hbm.at[src_v.at[j]], rows_v.at[b], sems[b])

    def wait(j, b):
      pltpu.make_async_copy(utab_hbm.at[src_v.at[j]], rows_v.at[b],
                            sems[b]).wait()

    for b in range(NBUF):
      fire(b, b)

    def body(m, carry):
      for b in range(NBUF):
        j = m * NBUF + b
        wait(j, b)
        pltpu.sync_copy(rows_v.at[b], acc.at[dst_v.at[j]], add=True)
        fire(j + NBUF, b)
      return carry

    lax.fori_loop(0, RPT // NBUF - 1, body, 0)
    for b in range(NBUF):
      j = RPT - NBUF + b
      wait(j, b)
      pltpu.sync_copy(rows_v.at[b], acc.at[dst_v.at[j]], add=True)

    plsc.subcore_barrier()
    pltpu.sync_copy(acc.at[pl.ds(s * STR, STR), :],
                    agg_hbm.at[c, pl.ds(s * STR, STR), :])

  return agg_kernel


_agg = _make_agg_kernel(64)


def _prep_body(degp_ref, x_ref, dinv_ref, u0_ref):
  d = degp_ref[:]
  deg = d[0, :, 0:1] + d[1, :, 0:1] + 1.0
  dinv = lax.rsqrt(deg)
  dinv_ref[:] = dinv
  u = x_ref[:] * dinv
  u0_ref[0] = u[:, : IN_CH // 2]
  u0_ref[1] = u[:, IN_CH // 2:]


def _prep_call(degp, x):
  return pl.pallas_call(
      _prep_body,
      grid=(GRID_TC,),
      in_specs=[
          pl.BlockSpec((NC, ROWS_TC, 16), lambda i: (0, i, 0)),
          pl.BlockSpec((ROWS_TC, IN_CH), lambda i: (i, 0)),
      ],
      out_specs=[
          pl.BlockSpec((ROWS_TC, 1), lambda i: (i, 0)),
          pl.BlockSpec((NC, ROWS_TC, IN_CH // 2), lambda i: (0, i, 0)),
      ],
      out_shape=[
          jax.ShapeDtypeStruct((N_NODES, 1), jnp.float32),
          jax.ShapeDtypeStruct((NC, N_NODES, IN_CH // 2), jnp.float32),
      ],
  )(degp, x)


def _make_layer_body(nplanes):
  def body(*refs):
    agg_refs = refs[:nplanes // 2]
    u_ref, dinv_ref, w_ref, b_ref, out_ref = refs[nplanes // 2:]
    parts = []
    for p in range(nplanes):
      parts.append(agg_refs[p // 2][p % 2] + u_ref[p])
    a = jnp.concatenate(parts, axis=1)
    dinv = dinv_ref[:]
    am = a * dinv
    h = jnp.dot(am, w_ref[:], preferred_element_type=jnp.float32) + b_ref[:]
    un = jnp.maximum(h, 0.0) * dinv
    for q in range(4):
      out_ref[q] = un[:, 64 * q: 64 * (q + 1)]
  return body


def _layer_call(aggs, u, dinv, w, b):
  nplanes = u.shape[0]
  din = 64 * nplanes
  agg_specs = [pl.BlockSpec((NC, ROWS_TC, 64), lambda i: (0, i, 0))
               for _ in aggs]
  return pl.pallas_call(
      _make_layer_body(nplanes),
      grid=(GRID_TC,),
      in_specs=agg_specs + [
          pl.BlockSpec((nplanes, ROWS_TC, 64), lambda i: (0, i, 0)),
          pl.BlockSpec((ROWS_TC, 1), lambda i: (i, 0)),
          pl.BlockSpec((din, HID), lambda i: (0, 0)),
          pl.BlockSpec((1, HID), lambda i: (0, 0)),
      ],
      out_specs=pl.BlockSpec((4, ROWS_TC, 64), lambda i: (0, i, 0)),
      out_shape=jax.ShapeDtypeStruct((4, N_NODES, 64), jnp.float32),
  )(*aggs, u, dinv, w, b.reshape(1, HID))


def _final_body(agga_ref, aggb_ref, u_ref, dinv_ref, w_ref, b_ref, batch_ref,
                wl1_ref, bl1_ref, wl2_ref, bl2_ref, out_ref, g_ref):
  i = pl.program_id(0)

  @pl.when(i == 0)
  def _():
    g_ref[:] = jnp.zeros_like(g_ref)

  agg_refs = (agga_ref, aggb_ref)
  parts = [agg_refs[p // 2][p % 2] + u_ref[p] for p in range(4)]
  a = jnp.concatenate(parts, axis=1)
  dinv = dinv_ref[:]
  am = a * dinv
  h3 = jnp.dot(am, w_ref[:], preferred_element_type=jnp.float32) + b_ref[:]
  ids = lax.broadcasted_iota(jnp.int32, (NUM_GRAPHS, ROWS_TC), 0)
  onehot = (ids == batch_ref[0]).astype(jnp.float32)
  g_ref[:] += jnp.dot(onehot, h3, preferred_element_type=jnp.float32)

  @pl.when(i == GRID_TC - 1)
  def _():
    y = jnp.maximum(
        jnp.dot(g_ref[:], wl1_ref[:], preferred_element_type=jnp.float32)
        + bl1_ref[:], 0.0)
    out_ref[:] = (
        jnp.dot(y, wl2_ref[:], preferred_element_type=jnp.float32)
        + bl2_ref[:])


def _final_call(agga, aggb, u, dinv, w2, b2, batch2, wl1, bl1, wl2, bl2):
  return pl.pallas_call(
      _final_body,
      grid=(GRID_TC,),
      in_specs=[
          pl.BlockSpec((NC, ROWS_TC, 64), lambda i: (0, i, 0)),
          pl.BlockSpec((NC, ROWS_TC, 64), lambda i: (0, i, 0)),
          pl.BlockSpec((4, ROWS_TC, 64), lambda i: (0, i, 0)),
          pl.BlockSpec((ROWS_TC, 1), lambda i: (i, 0)),
          pl.BlockSpec((HID, HID), lambda i: (0, 0)),
          pl.BlockSpec((1, HID), lambda i: (0, 0)),
          pl.BlockSpec((1, 1, ROWS_TC), lambda i: (i, 0, 0)),
          pl.BlockSpec((HID, 64), lambda i: (0, 0)),
          pl.BlockSpec((1, 64), lambda i: (0, 0)),
          pl.BlockSpec((64, 1), lambda i: (0, 0)),
          pl.BlockSpec((1, 1), lambda i: (0, 0)),
      ],
      out_specs=pl.BlockSpec((NUM_GRAPHS, 1), lambda i: (0, 0)),
      out_shape=jax.ShapeDtypeStruct((NUM_GRAPHS, 1), jnp.float32),
      scratch_shapes=[pltpu.VMEM((NUM_GRAPHS, HID), jnp.float32)],
  )(agga, aggb, u, dinv, w2, b2.reshape(1, HID), batch2,
    wl1, bl1.reshape(1, 64), wl2, bl2.reshape(1, 1))


def kernel(x, edge_index, batch, W0, b0, W1, b1, W2, b2, Wl1, bl1, Wl2, bl2):
  src = edge_index[0].astype(jnp.int32)
  dst = edge_index[1].astype(jnp.int32)
  src2 = jnp.stack([src, src + N_NODES]).reshape(NC, N_EDGES // K, K)
  dst_r = dst.reshape(N_EDGES // K, K)
  batch2 = batch.astype(jnp.int32).reshape(GRID_TC, 1, ROWS_TC)

  ones16 = jnp.ones((K, 16), jnp.float32)
  zeros16 = jnp.zeros((STR, 16), jnp.float32)
  zeros64 = jnp.zeros((STR, 64), jnp.float32)

  degp = _deg_kernel(dst_r, ones16, zeros16)
  dinv, u0 = _prep_call(degp, x)

  agg0 = _agg(u0.reshape(NC * N_NODES, 64), src2, dst_r, zeros64)
  u1 = _layer_call([agg0], u0, dinv, W0, b0)

  agg1a = _agg(u1[:2].reshape(NC * N_NODES, 64), src2, dst_r, zeros64)
  agg1b = _agg(u1[2:].reshape(NC * N_NODES, 64), src2, dst_r, zeros64)
  u2 = _layer_call([agg1a, agg1b], u1, dinv, W1, b1)

  agg2a = _agg(u2[:2].reshape(NC * N_NODES, 64), src2, dst_r, zeros64)
  agg2b = _agg(u2[2:].reshape(NC * N_NODES, 64), src2, dst_r, zeros64)
  return _final_call(agg2a, agg2b, u2, dinv, W2, b2, batch2,
                     Wl1, bl1, Wl2, bl2)
